# baseline (speedup 1.0000x reference)
import functools

import jax
import jax.numpy as jnp
from jax.experimental import pallas as pl
from jax.experimental.pallas import tpu as pltpu


def _act(th):
    t2 = th * th
    return (th + t2) / (1.0 + t2)


def _lattice_kernel(L, K, x_ref, wf_ref, wb_ref, bf_ref, bb_ref, out_ref,
                    hf_scr, hb_scr):
    k = pl.program_id(1)
    T = x_ref.shape[1]
    x = 0.5 * x_ref[0]
    bf = 0.5 * bf_ref[...]
    bb = 0.5 * bb_ref[...]
    hf_scr[pl.ds(k * T, T), :] = _act(jnp.tanh(
        jnp.dot(x, wf_ref[...], preferred_element_type=jnp.float32) + bf))
    hb_scr[pl.ds(k * T, T), :] = _act(jnp.tanh(
        jnp.dot(x, wb_ref[...], preferred_element_type=jnp.float32) + bb))

    @pl.when(k == K - 1)
    def _():
        H = wf_ref.shape[1]
        p = jax.lax.broadcasted_iota(jnp.int32, (L, 1), 0)
        zero = jnp.zeros((), jnp.float32)

        hf = hf_scr[...]
        f1 = hf[0:L]
        f2 = jnp.where(p >= 1, hf[L - 1:2 * L - 1], zero)
        f3 = jnp.where(p >= 2, hf[2 * L - 3:3 * L - 3], zero)
        f4 = jnp.where(p >= 3, hf[3 * L - 6:4 * L - 6], zero)
        cnt_f = jnp.minimum(p + 1, 4).astype(jnp.float32)
        out_ref[0, :, :H] = (f1 + f2 + f3 + f4) / cnt_f

        hb = hb_scr[...]
        b1 = hb[0:L]
        b2 = jnp.where(p <= L - 2, hb[L:2 * L], zero)
        b3 = jnp.where(p <= L - 3, hb[2 * L - 1:3 * L - 1], zero)
        b4 = jnp.where(p <= L - 4, hb[3 * L - 3:4 * L - 3], zero)
        cnt_b = jnp.minimum(L - p, 4).astype(jnp.float32)
        out_ref[0, :, H:] = (b1 + b2 + b3 + b4) / cnt_b


def kernel(edge_input, edge_begin, edge_end, W_ih_f, W_hh_f, b_f, W_ih_b, W_hh_b, b_b):
    del edge_begin, edge_end, W_hh_f, W_hh_b
    B, E, D = edge_input.shape
    H = W_ih_f.shape[1] // 4
    L = (E + 6) // 4
    K = 4
    T = ((E + K - 1) // K + 7) // 8 * 8
    S = max(4 * L - 3, K * T)

    out = pl.pallas_call(
        functools.partial(_lattice_kernel, L, K),
        grid=(B, K),
        in_specs=[
            pl.BlockSpec((1, T, D), lambda i, k: (i, k, 0)),
            pl.BlockSpec((D, H), lambda i, k: (0, 3)),
            pl.BlockSpec((D, H), lambda i, k: (0, 3)),
            pl.BlockSpec((1, H), lambda i, k: (0, 3)),
            pl.BlockSpec((1, H), lambda i, k: (0, 3)),
        ],
        out_specs=pl.BlockSpec((1, L, 2 * H), lambda i, k: (i, 0, 0)),
        out_shape=jax.ShapeDtypeStruct((B, L, 2 * H), jnp.float32),
        scratch_shapes=[
            pltpu.VMEM((S, H), jnp.float32),
            pltpu.VMEM((S, H), jnp.float32),
        ],
    )(edge_input, W_ih_f, W_ih_b, b_f[None, :], b_b[None, :])
    return out


# speedup vs baseline: 1.4260x; 1.4260x over previous
import functools

import jax
import jax.numpy as jnp
from jax.experimental import pallas as pl


def _act(o):
    t = jnp.tanh(0.5 * o)
    return t * (1.0 + t) / (1.0 + t * t)


def _lattice_kernel(L, x_ref, wf_ref, wb_ref, bf_ref, bb_ref, out_ref):
    x = x_ref[0]
    H = wf_ref.shape[1]
    p = jax.lax.broadcasted_iota(jnp.int32, (L, 1), 0)
    zero = jnp.zeros((), jnp.float32)

    hf = _act(jnp.dot(x, wf_ref[...], preferred_element_type=jnp.float32)
              + bf_ref[...])
    f1 = hf[0:L]
    f2 = jnp.where(p >= 1, hf[L - 1:2 * L - 1], zero)
    f3 = jnp.where(p >= 2, hf[2 * L - 3:3 * L - 3], zero)
    f4 = jnp.where(p >= 3, hf[3 * L - 6:4 * L - 6], zero)
    cnt_f = jnp.minimum(p + 1, 4).astype(jnp.float32)
    out_ref[0, :, :H] = (f1 + f2 + f3 + f4) / cnt_f

    hb = _act(jnp.dot(x, wb_ref[...], preferred_element_type=jnp.float32)
              + bb_ref[...])
    b1 = hb[0:L]
    b2 = jnp.where(p <= L - 2, hb[L:2 * L], zero)
    b3 = jnp.where(p <= L - 3, hb[2 * L - 1:3 * L - 1], zero)
    b4 = jnp.where(p <= L - 4, jnp.roll(hb[3 * L - 6:4 * L - 6], -3, axis=0), zero)
    cnt_b = jnp.minimum(L - p, 4).astype(jnp.float32)
    out_ref[0, :, H:] = (b1 + b2 + b3 + b4) / cnt_b


def kernel(edge_input, edge_begin, edge_end, W_ih_f, W_hh_f, b_f, W_ih_b, W_hh_b, b_b):
    del edge_begin, edge_end, W_hh_f, W_hh_b
    B, E, D = edge_input.shape
    H = W_ih_f.shape[1] // 4
    L = (E + 6) // 4

    out = pl.pallas_call(
        functools.partial(_lattice_kernel, L),
        grid=(B,),
        in_specs=[
            pl.BlockSpec((1, E, D), lambda i: (i, 0, 0)),
            pl.BlockSpec((D, H), lambda i: (0, 3)),
            pl.BlockSpec((D, H), lambda i: (0, 3)),
            pl.BlockSpec((1, H), lambda i: (0, 3)),
            pl.BlockSpec((1, H), lambda i: (0, 3)),
        ],
        out_specs=pl.BlockSpec((1, L, 2 * H), lambda i: (i, 0, 0)),
        out_shape=jax.ShapeDtypeStruct((B, L, 2 * H), jnp.float32),
    )(edge_input, W_ih_f, W_ih_b, b_f[None, :], b_b[None, :])
    return out


# speedup vs baseline: 1.4378x; 1.0083x over previous
import functools

import jax
import jax.numpy as jnp
from jax.experimental import pallas as pl
from jax.experimental.pallas import tpu as pltpu


def _act(th):
    t2 = th * th
    return (th + t2) / (1.0 + t2)


def _lattice_kernel(L, x1_ref, x2_ref, wf_ref, wb_ref, bf_ref, bb_ref,
                    out_ref, hf_scr, hb_scr):
    T = x1_ref.shape[1]
    H = wf_ref.shape[1]
    x1 = 0.5 * x1_ref[0]
    x2 = 0.5 * x2_ref[0]
    wf = wf_ref[...]
    wb = wb_ref[...]
    bf = 0.5 * bf_ref[...]
    bb = 0.5 * bb_ref[...]
    hf_scr[0:T, :] = _act(jnp.tanh(
        jnp.dot(x1, wf, preferred_element_type=jnp.float32) + bf))
    hf_scr[T:2 * T, :] = _act(jnp.tanh(
        jnp.dot(x2, wf, preferred_element_type=jnp.float32) + bf))
    hb_scr[0:T, :] = _act(jnp.tanh(
        jnp.dot(x1, wb, preferred_element_type=jnp.float32) + bb))
    hb_scr[T:2 * T, :] = _act(jnp.tanh(
        jnp.dot(x2, wb, preferred_element_type=jnp.float32) + bb))

    p = jax.lax.broadcasted_iota(jnp.int32, (L, 1), 0)
    zero = jnp.zeros((), jnp.float32)

    hf = hf_scr[...]
    f1 = hf[0:L]
    f2 = jnp.where(p >= 1, hf[L - 1:2 * L - 1], zero)
    f3 = jnp.where(p >= 2, hf[2 * L - 3:3 * L - 3], zero)
    f4 = jnp.where(p >= 3, hf[3 * L - 6:4 * L - 6], zero)
    cnt_f = jnp.minimum(p + 1, 4).astype(jnp.float32)
    out_ref[0, :, :H] = (f1 + f2 + f3 + f4) / cnt_f

    hb = hb_scr[...]
    b1 = hb[0:L]
    b2 = jnp.where(p <= L - 2, hb[L:2 * L], zero)
    b3 = jnp.where(p <= L - 3, hb[2 * L - 1:3 * L - 1], zero)
    b4 = jnp.where(p <= L - 4, hb[3 * L - 3:4 * L - 3], zero)
    cnt_b = jnp.minimum(L - p, 4).astype(jnp.float32)
    out_ref[0, :, H:] = (b1 + b2 + b3 + b4) / cnt_b


def kernel(edge_input, edge_begin, edge_end, W_ih_f, W_hh_f, b_f, W_ih_b, W_hh_b, b_b):
    del edge_begin, edge_end, W_hh_f, W_hh_b
    B, E, D = edge_input.shape
    H = W_ih_f.shape[1] // 4
    L = (E + 6) // 4
    T = (E + 15) // 16 * 8

    out = pl.pallas_call(
        functools.partial(_lattice_kernel, L),
        grid=(B,),
        in_specs=[
            pl.BlockSpec((1, T, D), lambda i: (i, 0, 0)),
            pl.BlockSpec((1, T, D), lambda i: (i, 1, 0)),
            pl.BlockSpec((D, H), lambda i: (0, 3)),
            pl.BlockSpec((D, H), lambda i: (0, 3)),
            pl.BlockSpec((1, H), lambda i: (0, 3)),
            pl.BlockSpec((1, H), lambda i: (0, 3)),
        ],
        out_specs=pl.BlockSpec((1, L, 2 * H), lambda i: (i, 0, 0)),
        out_shape=jax.ShapeDtypeStruct((B, L, 2 * H), jnp.float32),
        scratch_shapes=[
            pltpu.VMEM((2 * T, H), jnp.float32),
            pltpu.VMEM((2 * T, H), jnp.float32),
        ],
    )(edge_input, edge_input, W_ih_f, W_ih_b, b_f[None, :], b_b[None, :])
    return out


# speedup vs baseline: 1.4837x; 1.0319x over previous
import functools

import jax
import jax.numpy as jnp
from jax.experimental import pallas as pl
from jax.experimental.pallas import tpu as pltpu


def _act(th):
    t2 = th * th
    return (th + t2) / (1.0 + t2)


def _lattice_kernel(L, G, x1_ref, x2_ref, wf_ref, wb_ref, bf_ref, bb_ref,
                    out_ref, hf_scr, hb_scr):
    T = x1_ref.shape[1]
    H = wf_ref.shape[1]
    wf = wf_ref[...]
    wb = wb_ref[...]
    bf = 0.5 * bf_ref[...]
    bb = 0.5 * bb_ref[...]
    p = jax.lax.broadcasted_iota(jnp.int32, (L, 1), 0)
    zero = jnp.zeros((), jnp.float32)
    cnt_f = jnp.minimum(p + 1, 4).astype(jnp.float32)
    cnt_b = jnp.minimum(L - p, 4).astype(jnp.float32)

    for g in range(G):
        x1 = 0.5 * x1_ref[g]
        x2 = 0.5 * x2_ref[g]
        hf_scr[0:T, :] = _act(jnp.tanh(
            jnp.dot(x1, wf, preferred_element_type=jnp.float32) + bf))
        hf_scr[T:2 * T, :] = _act(jnp.tanh(
            jnp.dot(x2, wf, preferred_element_type=jnp.float32) + bf))
        hb_scr[0:T, :] = _act(jnp.tanh(
            jnp.dot(x1, wb, preferred_element_type=jnp.float32) + bb))
        hb_scr[T:2 * T, :] = _act(jnp.tanh(
            jnp.dot(x2, wb, preferred_element_type=jnp.float32) + bb))

        hf = hf_scr[...]
        f1 = hf[0:L]
        f2 = jnp.where(p >= 1, hf[L - 1:2 * L - 1], zero)
        f3 = jnp.where(p >= 2, hf[2 * L - 3:3 * L - 3], zero)
        f4 = jnp.where(p >= 3, hf[3 * L - 6:4 * L - 6], zero)
        out_ref[g, :, :H] = (f1 + f2 + f3 + f4) / cnt_f

        hb = hb_scr[...]
        b1 = hb[0:L]
        b2 = jnp.where(p <= L - 2, hb[L:2 * L], zero)
        b3 = jnp.where(p <= L - 3, hb[2 * L - 1:3 * L - 1], zero)
        b4 = jnp.where(p <= L - 4, hb[3 * L - 3:4 * L - 3], zero)
        out_ref[g, :, H:] = (b1 + b2 + b3 + b4) / cnt_b


def kernel(edge_input, edge_begin, edge_end, W_ih_f, W_hh_f, b_f, W_ih_b, W_hh_b, b_b):
    del edge_begin, edge_end, W_hh_f, W_hh_b
    B, E, D = edge_input.shape
    H = W_ih_f.shape[1] // 4
    L = (E + 6) // 4
    T = (E + 15) // 16 * 8
    G = 2

    out = pl.pallas_call(
        functools.partial(_lattice_kernel, L, G),
        grid=(B // G,),
        in_specs=[
            pl.BlockSpec((G, T, D), lambda i: (i, 0, 0)),
            pl.BlockSpec((G, T, D), lambda i: (i, 1, 0)),
            pl.BlockSpec((D, H), lambda i: (0, 3)),
            pl.BlockSpec((D, H), lambda i: (0, 3)),
            pl.BlockSpec((1, H), lambda i: (0, 3)),
            pl.BlockSpec((1, H), lambda i: (0, 3)),
        ],
        out_specs=pl.BlockSpec((G, L, 2 * H), lambda i: (i, 0, 0)),
        out_shape=jax.ShapeDtypeStruct((B, L, 2 * H), jnp.float32),
        scratch_shapes=[
            pltpu.VMEM((2 * T, H), jnp.float32),
            pltpu.VMEM((2 * T, H), jnp.float32),
        ],
    )(edge_input, edge_input, W_ih_f, W_ih_b, b_f[None, :], b_b[None, :])
    return out


# speedup vs baseline: 1.4852x; 1.0010x over previous
import functools

import jax
import jax.numpy as jnp
from jax.experimental import pallas as pl
from jax.experimental.pallas import tpu as pltpu


def _act(th):
    t2 = th * th
    return (th + t2) / (1.0 + t2)


def _lattice_kernel(L, G, x1_ref, x2_ref, x3_ref, x4_ref,
                    wf_ref, wb_ref, bf_ref, bb_ref, out_ref, hf_scr, hb_scr):
    T = x1_ref.shape[1]
    H = wf_ref.shape[1]
    wf = 0.5 * wf_ref[...]
    wb = 0.5 * wb_ref[...]
    bf = 0.5 * bf_ref[...]
    bb = 0.5 * bb_ref[...]
    p = jax.lax.broadcasted_iota(jnp.int32, (L, 1), 0)
    zero = jnp.zeros((), jnp.float32)
    cnt_f = jnp.minimum(p + 1, 4).astype(jnp.float32)
    cnt_b = jnp.minimum(L - p, 4).astype(jnp.float32)

    for g in range(G):
        for q, x_ref in enumerate((x1_ref, x2_ref, x3_ref, x4_ref)):
            x = x_ref[g]
            hf_scr[q * T:(q + 1) * T, :] = _act(jnp.tanh(
                jnp.dot(x, wf, preferred_element_type=jnp.float32) + bf))
            hb_scr[q * T:(q + 1) * T, :] = _act(jnp.tanh(
                jnp.dot(x, wb, preferred_element_type=jnp.float32) + bb))

        hf = hf_scr[...]
        f1 = hf[0:L]
        f2 = jnp.where(p >= 1, hf[L - 1:2 * L - 1], zero)
        f3 = jnp.where(p >= 2, hf[2 * L - 3:3 * L - 3], zero)
        f4 = jnp.where(p >= 3, hf[3 * L - 6:4 * L - 6], zero)
        out_ref[g, :, :H] = (f1 + f2 + f3 + f4) / cnt_f

        hb = hb_scr[...]
        b1 = hb[0:L]
        b2 = jnp.where(p <= L - 2, hb[L:2 * L], zero)
        b3 = jnp.where(p <= L - 3, hb[2 * L - 1:3 * L - 1], zero)
        b4 = jnp.where(p <= L - 4, hb[3 * L - 3:4 * L - 3], zero)
        out_ref[g, :, H:] = (b1 + b2 + b3 + b4) / cnt_b


def kernel(edge_input, edge_begin, edge_end, W_ih_f, W_hh_f, b_f, W_ih_b, W_hh_b, b_b):
    del edge_begin, edge_end, W_hh_f, W_hh_b
    B, E, D = edge_input.shape
    H = W_ih_f.shape[1] // 4
    L = (E + 6) // 4
    T = (E + 31) // 32 * 8
    G = 2

    xspec = lambda q: pl.BlockSpec((G, T, D), lambda i, q=q: (i, q, 0))
    out = pl.pallas_call(
        functools.partial(_lattice_kernel, L, G),
        grid=(B // G,),
        in_specs=[
            xspec(0), xspec(1), xspec(2), xspec(3),
            pl.BlockSpec((D, H), lambda i: (0, 3)),
            pl.BlockSpec((D, H), lambda i: (0, 3)),
            pl.BlockSpec((1, H), lambda i: (0, 3)),
            pl.BlockSpec((1, H), lambda i: (0, 3)),
        ],
        out_specs=pl.BlockSpec((G, L, 2 * H), lambda i: (i, 0, 0)),
        out_shape=jax.ShapeDtypeStruct((B, L, 2 * H), jnp.float32),
        scratch_shapes=[
            pltpu.VMEM((4 * T, H), jnp.float32),
            pltpu.VMEM((4 * T, H), jnp.float32),
        ],
    )(edge_input, edge_input, edge_input, edge_input,
      W_ih_f, W_ih_b, b_f[None, :], b_b[None, :])
    return out
